# baseline (speedup 1.0000x reference)
import functools

import jax
import jax.numpy as jnp
from jax import lax
from jax.experimental import pallas as pl
from jax.experimental.pallas import tpu as pltpu
from jax.experimental.pallas import tpu_sc as plsc

COULOMB = 322.0637
_NC = 2
_NS = 16
_QOFF = 0
_MASKOFF = 656


def _sc_gather(n_rows, rows_per_worker, n_q_chunks, table_hbm, idx_hbm,
               out_hbm, idx_v, rows_v, sem):
    wid = lax.axis_index("s") * _NC + lax.axis_index("c")
    n_workers = n_rows // rows_per_worker

    @pl.when(wid < n_workers)
    def _():
        base = wid * rows_per_worker
        pltpu.sync_copy(idx_hbm.at[pl.ds(base, rows_per_worker)], idx_v)
        pltpu.async_copy(table_hbm.at[idx_v], rows_v, sem).wait()
        for i in range(rows_per_worker):
            for k in range(n_q_chunks):
                q = rows_v[i, pl.ds(_QOFF + 16 * k, 16)]
                m = rows_v[i, pl.ds(_MASKOFF + 16 * k, 16)]
                rows_v[i, pl.ds(_QOFF + 16 * k, 16)] = q * m
        pltpu.sync_copy(rows_v, out_hbm.at[pl.ds(base, rows_per_worker)])


def _dense_body(A, N, K, G, gp, qcol_in, qrow_in, ucol_in, vown_in, ibown_in,
                intra_in, xi_in, xiT_in, xT_in, qrow_j_in, v_row_in, bexp_in,
                out_ref):
    bg = pl.program_id(1)
    J = N // K

    D = gp[0:1, 0:1]
    D0 = gp[0:1, 1:2]
    S = gp[0:1, 2:3]
    mind = gp[0:1, 3:4]
    maxd = gp[0:1, 4:5]

    def e_coul(x):
        xs = x * S
        eps = D - 0.5 * (D - D0) * (2.0 + 2.0 * xs + xs * xs) * jnp.exp(-xs)
        return COULOMB / (eps * x)

    e_shift = e_coul(maxd)

    def energy(d2):
        d = jnp.sqrt(jnp.maximum(d2, 1.0))
        return e_coul(jnp.clip(d, mind, maxd)) - e_shift

    xTs = []
    xj2s = []
    for j in range(J):
        xT = xT_in[0, :, j, :]
        xTs.append(xT)
        xj2s.append(xT[0:1, :] ** 2 + xT[1:2, :] ** 2 + xT[2:3, :] ** 2)

    part = jnp.zeros((1, 1), jnp.float32)
    for g in range(G):
        xi = xi_in[0, g]
        xi2 = (xi[:, 0:1] ** 2 + xi[:, 1:2] ** 2 + xi[:, 2:3] ** 2)
        u = ucol_in[0, g]
        u0 = u[0]
        u1 = u[1]

        acc = jnp.zeros((A, K), jnp.float32)
        for j in range(J):
            dot = jnp.dot(xi, xTs[j],
                          preferred_element_type=jnp.float32)
            d2 = (xj2s[j] - 2.0 * dot) + xi2
            e = energy(d2)

            v = v_row_in[0, :, j, :]
            bexp = bexp_in[0, g, :, j, :]
            m0 = jnp.minimum(bexp[0:1] + v[0:1], bexp[1:2] + v[1:2])
            m1 = jnp.minimum(bexp[2:3] + v[0:1], bexp[3:4] + v[1:2])
            sep = jnp.minimum(u0 + m0, u1 + m1)
            w = jnp.clip(0.8 * sep.astype(jnp.float32) - 3.0, 0.0, 1.0)

            acc = acc + e * (w * qrow_j_in[0, 0:1, j, :])

        qm_col = qcol_in[0, g]
        inter = jnp.sum(acc * qm_col, keepdims=True)

        xiT = xiT_in[0, g]
        doto = jnp.dot(xi, xiT, preferred_element_type=jnp.float32)
        xj2o = xiT[0:1, :] ** 2 + xiT[1:2, :] ** 2 + xiT[2:3, :] ** 2
        eo = energy((xj2o - 2.0 * doto) + xi2)

        vown = vown_in[0, g]
        ibo = ibown_in[0, g]
        m0o = jnp.minimum(ibo[0:1] + vown[0:1, :], ibo[1:2] + vown[1:2, :])
        m1o = jnp.minimum(ibo[2:3] + vown[0:1, :], ibo[3:4] + vown[1:2, :])
        sepo = jnp.minimum(u0 + m0o, u1 + m1o)
        wo = jnp.clip(0.8 * sepo.astype(jnp.float32) - 3.0, 0.0, 1.0)
        sepi = intra_in[0, g]
        wi = jnp.clip(0.8 * sepi.astype(jnp.float32) - 3.0, 0.0, 1.0)

        qm_rowi = qrow_in[0, g]
        own = jnp.sum(eo * (wi - wo) * qm_rowi * qm_col, keepdims=True)

        part = part + 0.5 * (inter + own)

    @pl.when(bg == 0)
    def _():
        out_ref[0] = jnp.zeros((1, 1), jnp.float32)

    out_ref[0] += part


def kernel(coords, pose_stack_block_coord_offset, pose_stack_block_types,
           pose_stack_min_block_bondsep, pose_stack_inter_block_bondsep,
           bt_n_atoms, bt_partial_charge, bt_n_interblock_bonds,
           bt_atoms_forming_chemical_bonds, bt_inter_repr_path_distance,
           bt_intra_repr_path_distance, global_params):
    P, B = pose_stack_block_types.shape
    T, A = bt_partial_charge.shape
    C = bt_inter_repr_path_distance.shape[1]
    N = B * A

    bt = pose_stack_block_types.astype(jnp.int32)
    n_at = bt_n_atoms.astype(jnp.int32)

    conn_bits = lax.bitcast_convert_type(
        bt_inter_repr_path_distance.astype(jnp.int32).reshape(T, C * A),
        jnp.float32)
    intra_bits = lax.bitcast_convert_type(
        bt_intra_repr_path_distance.astype(jnp.int32).reshape(T, A * A),
        jnp.float32)
    mask_f = (jnp.arange(A, dtype=jnp.int32)[None, :]
              < n_at[:, None]).astype(jnp.float32)
    W = 768
    pad8 = jnp.zeros((T, _MASKOFF - (A + C * A + A * A)), jnp.float32)
    padE = jnp.zeros((T, W - _MASKOFF - A), jnp.float32)
    table = jnp.concatenate(
        [bt_partial_charge, conn_bits, intra_bits, pad8, mask_f, padE],
        axis=1)
    R = P * B
    rpw = R // 16

    gathered = pl.kernel(
        functools.partial(_sc_gather, R, rpw, (A + 15) // 16),
        out_type=jax.ShapeDtypeStruct((R, W), jnp.float32),
        mesh=plsc.VectorSubcoreMesh(core_axis_name="c", subcore_axis_name="s"),
        scratch_types=[
            pltpu.VMEM((rpw,), jnp.int32),
            pltpu.VMEM((rpw, W), jnp.float32),
            pltpu.SemaphoreType.DMA,
        ],
    )(table, bt.reshape(R))

    K = 128
    J = N // K
    q_pb = gathered[:, 0:A]
    v_pb = lax.bitcast_convert_type(
        gathered[:, A:A + C * A], jnp.int32).reshape(P, B, C, A)
    intra_pb = lax.bitcast_convert_type(
        gathered[:, A + C * A:A + C * A + A * A],
        jnp.int32).reshape(P, B, A, A)
    qrow_j = q_pb.reshape(P, 1, J, K)
    q_col = q_pb.reshape(P, B, A, 1)
    q_row = q_pb.reshape(P, B, 1, A)
    u_col = v_pb.reshape(P, B, C, A, 1)
    v_row = v_pb.transpose(0, 2, 1, 3).reshape(P, C, J, K)
    ib = pose_stack_inter_block_bondsep.astype(jnp.int32)
    t = ib.transpose(0, 1, 3, 4, 2).reshape(P, B, C * C, B)
    b_exp = jnp.broadcast_to(t[..., None], (P, B, C * C, B, A)).reshape(
        P, B, C * C, J, K)
    bidx = jnp.arange(B)
    ib_own = ib[:, bidx, bidx].transpose(0, 1, 2, 3).reshape(
        P, B, C * C, 1)

    coords_pb4 = coords.reshape(P, B, A, 3)
    xi8 = jnp.concatenate(
        [coords_pb4, jnp.zeros((P, B, A, 5), coords.dtype)], axis=3)
    xiT8 = xi8.transpose(0, 1, 3, 2)
    coords_T8 = xi8.reshape(P, N, 8).transpose(0, 2, 1).reshape(P, 8, J, K)
    gp = global_params.astype(jnp.float32)

    G = 8
    grid_b = pl.GridSpec(
        grid=(P, B // G),
        in_specs=[
            pl.BlockSpec((1, 5), lambda p, b: (0, 0)),
            pl.BlockSpec((1, G, A, 1), lambda p, b: (p, b, 0, 0)),
            pl.BlockSpec((1, G, 1, A), lambda p, b: (p, b, 0, 0)),
            pl.BlockSpec((1, G, C, A, 1), lambda p, b: (p, b, 0, 0, 0)),
            pl.BlockSpec((1, G, C, A), lambda p, b: (p, b, 0, 0)),
            pl.BlockSpec((1, G, C * C, 1), lambda p, b: (p, b, 0, 0)),
            pl.BlockSpec((1, G, A, A), lambda p, b: (p, b, 0, 0)),
            pl.BlockSpec((1, G, A, 8), lambda p, b: (p, b, 0, 0)),
            pl.BlockSpec((1, G, 8, A), lambda p, b: (p, b, 0, 0)),
            pl.BlockSpec((1, 8, J, K), lambda p, b: (p, 0, 0, 0)),
            pl.BlockSpec((1, 1, J, K), lambda p, b: (p, 0, 0, 0)),
            pl.BlockSpec((1, C, J, K), lambda p, b: (p, 0, 0, 0)),
            pl.BlockSpec((1, G, C * C, J, K), lambda p, b: (p, b, 0, 0, 0)),
        ],
        out_specs=pl.BlockSpec((1, 1, 1), lambda p, b: (p, 0, 0)),
    )
    out3 = pl.pallas_call(
        functools.partial(_dense_body, A, N, K, G),
        grid_spec=grid_b,
        out_shape=jax.ShapeDtypeStruct((P, 1, 1), jnp.float32),
        compiler_params=pltpu.CompilerParams(
            dimension_semantics=("parallel", "arbitrary")),
    )(gp, q_col, q_row, u_col, v_pb, ib_own, intra_pb,
      xi8, xiT8, coords_T8, qrow_j, v_row, b_exp)

    return out3.reshape(P)


# speedup vs baseline: 1.1217x; 1.1217x over previous
import functools

import jax
import jax.numpy as jnp
from jax import lax
from jax.experimental import pallas as pl
from jax.experimental.pallas import tpu as pltpu
from jax.experimental.pallas import tpu_sc as plsc

COULOMB = 322.0637
_NC = 2
_NS = 16


def _sc_gather(n_rows, rows_per_worker, table_hbm, idx_hbm, out_hbm,
               idx_v, rows_v, sem):
    wid = lax.axis_index("s") * _NC + lax.axis_index("c")
    n_workers = n_rows // rows_per_worker

    @pl.when(wid < n_workers)
    def _():
        base = wid * rows_per_worker
        pltpu.sync_copy(idx_hbm.at[pl.ds(base, rows_per_worker)], idx_v)
        pltpu.async_copy(table_hbm.at[idx_v], rows_v, sem).wait()
        pltpu.sync_copy(rows_v, out_hbm.at[pl.ds(base, rows_per_worker)])


def _dense_body(A, N, K, G, bt_ref, n_ref, gp, qcol_in, qrow_in, ucol_in,
                intra_in, xi_in, xiT_in, xT_in, qrow_j_in, nrow_j_in,
                arow_j_in, v_row_in, bexp_in, out_ref):
    p = pl.program_id(0)
    bg = pl.program_id(1)
    J = N // K

    D = gp[0:1, 0:1]
    D0 = gp[0:1, 1:2]
    S = gp[0:1, 2:3]
    mind = gp[0:1, 3:4]
    maxd = gp[0:1, 4:5]

    def e_coul(x):
        xs = x * S
        eps = D - 0.5 * (D - D0) * (2.0 + 2.0 * xs + xs * xs) * jnp.exp(-xs)
        return COULOMB / (eps * x)

    e_shift = e_coul(maxd)

    part = jnp.zeros((1, 1), jnp.float32)
    for g in range(G):
        n = n_ref[bt_ref[p, bg * G + g]]
        xi = xi_in[0, g]
        x0 = xi[:, 0:1]
        x1 = xi[:, 1:2]
        x2 = xi[:, 2:3]
        u = ucol_in[0, g]
        u0 = u[0]
        u1 = u[1]
        jstart = (bg * G + g) * A

        acc = jnp.zeros((A, K), jnp.float32)
        for j in range(J):
            xT = xT_in[0, :, j, :]
            d2 = ((x0 - xT[0:1, :]) ** 2
                  + (x1 - xT[1:2, :]) ** 2
                  + (x2 - xT[2:3, :]) ** 2)
            d = jnp.sqrt(jnp.maximum(d2, 1.0))
            e = e_coul(jnp.clip(d, mind, maxd)) - e_shift

            v = v_row_in[0, :, j, :]
            bexp = bexp_in[0, g, :, j, :]
            m0 = jnp.minimum(bexp[0:1] + v[0:1], bexp[1:2] + v[1:2])
            m1 = jnp.minimum(bexp[2:3] + v[0:1], bexp[3:4] + v[1:2])
            sep = jnp.minimum(u0 + m0, u1 + m1)
            w = jnp.clip(0.8 * sep.astype(jnp.float32) - 3.0, 0.0, 1.0)

            lane_j = j * K + jax.lax.broadcasted_iota(jnp.int32, (1, K), 1)
            notsame = (lane_j < jstart) | (lane_j >= jstart + A)
            jmask = (arow_j_in[0:1, j, :] < nrow_j_in[0, 0:1, j, :]) & notsame
            row_eff = qrow_j_in[0, 0:1, j, :] * jmask.astype(jnp.float32)

            acc = acc + e * (w * row_eff)

        col_a = jax.lax.broadcasted_iota(jnp.int32, (A, 1), 0)
        qm_col = qcol_in[0, g] * (col_a < n).astype(jnp.float32)

        inter = jnp.sum(acc * qm_col, keepdims=True)

        xiT = xiT_in[0, g]
        d2i = ((xi[:, 0:1] - xiT[0:1, :]) ** 2
               + (xi[:, 1:2] - xiT[1:2, :]) ** 2
               + (xi[:, 2:3] - xiT[2:3, :]) ** 2)
        di = jnp.sqrt(jnp.maximum(d2i, 1.0))
        ei = e_coul(jnp.clip(di, mind, maxd)) - e_shift
        sepi = intra_in[0, g]
        wi = jnp.clip(0.8 * sepi.astype(jnp.float32) - 3.0, 0.0, 1.0)
        row_a = jax.lax.broadcasted_iota(jnp.int32, (1, A), 1)
        qm_rowi = qrow_in[0, g] * (row_a < n).astype(jnp.float32)
        intra = jnp.sum(ei * wi * qm_rowi * qm_col, keepdims=True)

        part = part + 0.5 * (inter + intra)

    @pl.when(bg == 0)
    def _():
        out_ref[0] = jnp.zeros((1, 1), jnp.float32)

    out_ref[0] += part


def kernel(coords, pose_stack_block_coord_offset, pose_stack_block_types,
           pose_stack_min_block_bondsep, pose_stack_inter_block_bondsep,
           bt_n_atoms, bt_partial_charge, bt_n_interblock_bonds,
           bt_atoms_forming_chemical_bonds, bt_inter_repr_path_distance,
           bt_intra_repr_path_distance, global_params):
    P, B = pose_stack_block_types.shape
    T, A = bt_partial_charge.shape
    C = bt_inter_repr_path_distance.shape[1]
    N = B * A

    bt = pose_stack_block_types.astype(jnp.int32)
    n_at = bt_n_atoms.astype(jnp.int32)

    conn_bits = lax.bitcast_convert_type(
        bt_inter_repr_path_distance.astype(jnp.int32).reshape(T, C * A),
        jnp.float32)
    intra_bits = lax.bitcast_convert_type(
        bt_intra_repr_path_distance.astype(jnp.int32).reshape(T, A * A),
        jnp.float32)
    n_f = jnp.broadcast_to(n_at.astype(jnp.float32)[:, None], (T, 8))
    W = 768
    pad = jnp.zeros((T, W - (A + C * A + A * A + 8)), jnp.float32)
    table = jnp.concatenate(
        [bt_partial_charge, conn_bits, intra_bits, n_f, pad], axis=1)
    R = P * B
    rpw = R // 16

    gathered = pl.kernel(
        functools.partial(_sc_gather, R, rpw),
        out_type=jax.ShapeDtypeStruct((R, W), jnp.float32),
        mesh=plsc.VectorSubcoreMesh(core_axis_name="c", subcore_axis_name="s"),
        scratch_types=[
            pltpu.VMEM((rpw,), jnp.int32),
            pltpu.VMEM((rpw, W), jnp.float32),
            pltpu.SemaphoreType.DMA,
        ],
    )(table, bt.reshape(R))

    K = 128
    J = N // K
    q_pb = gathered[:, 0:A]
    v_pb = lax.bitcast_convert_type(
        gathered[:, A:A + C * A], jnp.int32).reshape(P, B, C, A)
    intra_pb = lax.bitcast_convert_type(
        gathered[:, A + C * A:A + C * A + A * A],
        jnp.int32).reshape(P, B, A, A)
    n_row = jnp.broadcast_to(
        gathered[:, A + C * A + A * A:A + C * A + A * A + 1].reshape(P, B, 1),
        (P, B, A)).reshape(P, 1, J, K)
    qrow_j = q_pb.reshape(P, 1, J, K)
    arow_j = jnp.broadcast_to(
        jnp.arange(A, dtype=jnp.float32)[None, :], (B, A)).reshape(1, J, K)
    q_col = q_pb.reshape(P, B, A, 1)
    q_row = q_pb.reshape(P, B, 1, A)
    u_col = v_pb.reshape(P, B, C, A, 1)
    v_row = v_pb.transpose(0, 2, 1, 3).reshape(P, C, J, K)
    ib = pose_stack_inter_block_bondsep.astype(jnp.int32)
    t = ib.transpose(0, 1, 3, 4, 2).reshape(P, B, C * C, B)
    b_exp = jnp.broadcast_to(t[..., None], (P, B, C * C, B, A)).reshape(
        P, B, C * C, J, K)

    coords_pb4 = coords.reshape(P, B, A, 3)
    coords_bT = coords_pb4.transpose(0, 1, 3, 2)
    coords_T = coords.transpose(0, 2, 1).reshape(P, 3, J, K)
    gp = global_params.astype(jnp.float32)

    G = 8
    grid_b = pltpu.PrefetchScalarGridSpec(
        num_scalar_prefetch=2,
        grid=(P, B // G),
        in_specs=[
            pl.BlockSpec((1, 5), lambda p, b, btr, nr: (0, 0)),
            pl.BlockSpec((1, G, A, 1), lambda p, b, btr, nr: (p, b, 0, 0)),
            pl.BlockSpec((1, G, 1, A), lambda p, b, btr, nr: (p, b, 0, 0)),
            pl.BlockSpec((1, G, C, A, 1),
                         lambda p, b, btr, nr: (p, b, 0, 0, 0)),
            pl.BlockSpec((1, G, A, A), lambda p, b, btr, nr: (p, b, 0, 0)),
            pl.BlockSpec((1, G, A, 3), lambda p, b, btr, nr: (p, b, 0, 0)),
            pl.BlockSpec((1, G, 3, A), lambda p, b, btr, nr: (p, b, 0, 0)),
            pl.BlockSpec((1, 3, J, K), lambda p, b, btr, nr: (p, 0, 0, 0)),
            pl.BlockSpec((1, 1, J, K), lambda p, b, btr, nr: (p, 0, 0, 0)),
            pl.BlockSpec((1, 1, J, K), lambda p, b, btr, nr: (p, 0, 0, 0)),
            pl.BlockSpec((1, J, K), lambda p, b, btr, nr: (0, 0, 0)),
            pl.BlockSpec((1, C, J, K), lambda p, b, btr, nr: (p, 0, 0, 0)),
            pl.BlockSpec((1, G, C * C, J, K),
                         lambda p, b, btr, nr: (p, b, 0, 0, 0)),
        ],
        out_specs=pl.BlockSpec((1, 1, 1), lambda p, b, btr, nr: (p, 0, 0)),
    )
    out3 = pl.pallas_call(
        functools.partial(_dense_body, A, N, K, G),
        grid_spec=grid_b,
        out_shape=jax.ShapeDtypeStruct((P, 1, 1), jnp.float32),
        compiler_params=pltpu.CompilerParams(
            dimension_semantics=("parallel", "arbitrary")),
    )(bt, n_at, gp, q_col, q_row, u_col, intra_pb,
      coords_pb4, coords_bT, coords_T, qrow_j, n_row, arow_j, v_row, b_exp)

    return out3.reshape(P)
